# baseline (device time: 49214 ns/iter reference)
import jax
import jax.numpy as jnp
from jax import lax
from jax.experimental import pallas as pl
from jax.experimental.pallas import tpu as pltpu

N_DEV = 8
M = 1024
N = 1024
CHUNK = M // N_DEV
NSTREAM = 16
BAND = CHUNK // NSTREAM
SIGMA = tuple(1 if k % 2 == 0 else -1 for k in range(NSTREAM))
BANDOF = tuple(
    k // 2 if k % 2 == 0 else NSTREAM // 2 + k // 2 for k in range(NSTREAM)
)
SLOT_DEPTH = 3
SEM_DEPTH = 2
T_RS = N_DEV - 1
T_TOT = 2 * (N_DEV - 1)


def kernel(x, w_mat):
    def body(
        x_ref,
        w_ref,
        out_ref,
        pacc_ref,
        sendbuf_ref,
        slots_ref,
        send_sems,
        recv_sems,
    ):
        my = lax.axis_index("i")

        def mod8(v):
            return lax.rem(v + 16, N_DEV)

        def perm(j):
            return jnp.where(j < 4, j, 11 - j)

        my_r = perm(my)
        tgt = [perm(mod8(my_r + SIGMA[k])) for k in range(NSTREAM)]

        def band_row(ring_j, k):
            return perm(ring_j) * CHUNK + BANDOF[k] * BAND

        def gemm_chunk(ring_j):
            r = perm(ring_j) * CHUNK
            pacc_ref[pl.ds(r, CHUNK), :] = jnp.dot(
                x_ref[pl.ds(r, CHUNK), :],
                w_ref[...],
                preferred_element_type=jnp.float32,
            )

        barrier = pltpu.get_barrier_semaphore()
        for nbr in (tgt[0], tgt[1]):
            pl.semaphore_signal(
                barrier,
                inc=1,
                device_id=(nbr,),
                device_id_type=pl.DeviceIdType.MESH,
            )
        gemm_chunk(my_r)
        pl.semaphore_wait(barrier, 2)

        def make(k, t):
            sig = SIGMA[k]
            if t < T_RS:
                src = sendbuf_ref.at[k]
                dst = slots_ref.at[k, t % SLOT_DEPTH]
            else:
                c = mod8(my_r + sig * (N_DEV - t))
                r = band_row(c, k)
                src = out_ref.at[pl.ds(r, BAND), :]
                dst = out_ref.at[pl.ds(r, BAND), :]
            return pltpu.make_async_remote_copy(
                src_ref=src,
                dst_ref=dst,
                send_sem=send_sems.at[k, t % SEM_DEPTH],
                recv_sem=recv_sems.at[k, t % SEM_DEPTH],
                device_id=(tgt[k],),
                device_id_type=pl.DeviceIdType.MESH,
            )

        for t in range(T_TOT):
            for k in range(NSTREAM):
                sig = SIGMA[k]
                if t > 0:
                    make(k, t - 1).wait()
                if t == 0:
                    sendbuf_ref[k] = pacc_ref[pl.ds(band_row(my_r, k), BAND), :]
                elif t < T_RS:
                    c = mod8(my_r - sig * t)
                    sendbuf_ref[k] = (
                        pacc_ref[pl.ds(band_row(c, k), BAND), :]
                        + slots_ref[k, (t - 1) % SLOT_DEPTH]
                    )
                elif t == T_RS:
                    c = mod8(my_r + sig)
                    r = band_row(c, k)
                    acc = (
                        pacc_ref[pl.ds(r, BAND), :]
                        + slots_ref[k, (t - 1) % SLOT_DEPTH]
                    )
                    y = acc * jax.nn.sigmoid(acc)
                    out_ref[pl.ds(r, BAND), :] = y
                make(k, t).start()
            if t < 4:
                offs = [t + 1, -(t + 1)] if t < 3 else [4]
                for off in offs:
                    gemm_chunk(mod8(my_r + off))

        for k in range(NSTREAM):
            make(k, T_TOT - 1).wait()

    return pl.pallas_call(
        body,
        out_shape=jax.ShapeDtypeStruct((M, N), jnp.float32),
        in_specs=[
            pl.BlockSpec(memory_space=pltpu.VMEM),
            pl.BlockSpec(memory_space=pltpu.VMEM),
        ],
        out_specs=pl.BlockSpec(memory_space=pltpu.VMEM),
        scratch_shapes=[
            pltpu.VMEM((M, N), jnp.float32),
            pltpu.VMEM((NSTREAM, BAND, N), jnp.float32),
            pltpu.VMEM((NSTREAM, SLOT_DEPTH, BAND, N), jnp.float32),
            pltpu.SemaphoreType.DMA((NSTREAM, SEM_DEPTH)),
            pltpu.SemaphoreType.DMA((NSTREAM, SEM_DEPTH)),
        ],
        compiler_params=pltpu.CompilerParams(collective_id=0),
    )(x, w_mat)


# device time: 48954 ns/iter; 1.0053x vs baseline; 1.0053x over previous
import jax
import jax.numpy as jnp
from jax import lax
from jax.experimental import pallas as pl
from jax.experimental.pallas import tpu as pltpu

N_DEV = 8
M = 1024
N = 1024
CHUNK = M // N_DEV
NSTREAM = 8
BAND = CHUNK // NSTREAM
SIGMA = (1, -1, 1, -1, 1, -1, 1, -1)
BANDOF = (0, 4, 1, 5, 2, 6, 3, 7)
SLOT_DEPTH = 3
SEM_DEPTH = 2
T_RS = N_DEV - 1
T_TOT = 2 * (N_DEV - 1)


def kernel(x, w_mat):
    def body(
        x_ref,
        w_ref,
        out_ref,
        pacc_ref,
        sendbuf_ref,
        slots_ref,
        send_sems,
        recv_sems,
    ):
        my = lax.axis_index("i")

        def mod8(v):
            return lax.rem(v + 16, N_DEV)

        def perm(j):
            return jnp.where(j < 4, j, 11 - j)

        my_r = perm(my)
        tgt = [perm(mod8(my_r + SIGMA[k])) for k in range(NSTREAM)]

        def band_row(ring_j, k):
            return perm(ring_j) * CHUNK + BANDOF[k] * BAND

        def gemm_chunk(ring_j):
            r = perm(ring_j) * CHUNK
            pacc_ref[pl.ds(r, CHUNK), :] = jnp.dot(
                x_ref[pl.ds(r, CHUNK), :],
                w_ref[...],
                preferred_element_type=jnp.float32,
            )

        barrier = pltpu.get_barrier_semaphore()
        for nbr in (tgt[0], tgt[1]):
            pl.semaphore_signal(
                barrier,
                inc=1,
                device_id=(nbr,),
                device_id_type=pl.DeviceIdType.MESH,
            )
        gemm_chunk(my_r)
        pl.semaphore_wait(barrier, 2)

        def make(k, t):
            sig = SIGMA[k]
            if t < T_RS:
                src = sendbuf_ref.at[k]
                dst = slots_ref.at[k, t % SLOT_DEPTH]
            else:
                c = mod8(my_r + sig * (N_DEV - t))
                r = band_row(c, k)
                src = out_ref.at[pl.ds(r, BAND), :]
                dst = out_ref.at[pl.ds(r, BAND), :]
            return pltpu.make_async_remote_copy(
                src_ref=src,
                dst_ref=dst,
                send_sem=send_sems.at[k, t % SEM_DEPTH],
                recv_sem=recv_sems.at[k, t % SEM_DEPTH],
                device_id=(tgt[k],),
                device_id_type=pl.DeviceIdType.MESH,
            )

        for t in range(T_TOT):
            for k in range(NSTREAM):
                sig = SIGMA[k]
                if t > 0:
                    make(k, t - 1).wait()
                if t == 0:
                    sendbuf_ref[k] = pacc_ref[pl.ds(band_row(my_r, k), BAND), :]
                elif t < T_RS:
                    c = mod8(my_r - sig * t)
                    sendbuf_ref[k] = (
                        pacc_ref[pl.ds(band_row(c, k), BAND), :]
                        + slots_ref[k, (t - 1) % SLOT_DEPTH]
                    )
                elif t == T_RS:
                    c = mod8(my_r + sig)
                    r = band_row(c, k)
                    acc = (
                        pacc_ref[pl.ds(r, BAND), :]
                        + slots_ref[k, (t - 1) % SLOT_DEPTH]
                    )
                    y = acc
                    out_ref[pl.ds(r, BAND), :] = y
                make(k, t).start()
            if t < 4:
                offs = [t + 1, -(t + 1)] if t < 3 else [4]
                for off in offs:
                    gemm_chunk(mod8(my_r + off))

        for k in range(NSTREAM):
            make(k, T_TOT - 1).wait()

    return pl.pallas_call(
        body,
        out_shape=jax.ShapeDtypeStruct((M, N), jnp.float32),
        in_specs=[
            pl.BlockSpec(memory_space=pltpu.VMEM),
            pl.BlockSpec(memory_space=pltpu.VMEM),
        ],
        out_specs=pl.BlockSpec(memory_space=pltpu.VMEM),
        scratch_shapes=[
            pltpu.VMEM((M, N), jnp.float32),
            pltpu.VMEM((NSTREAM, BAND, N), jnp.float32),
            pltpu.VMEM((NSTREAM, SLOT_DEPTH, BAND, N), jnp.float32),
            pltpu.SemaphoreType.DMA((NSTREAM, SEM_DEPTH)),
            pltpu.SemaphoreType.DMA((NSTREAM, SEM_DEPTH)),
        ],
        compiler_params=pltpu.CompilerParams(collective_id=0),
    )(x, w_mat)


# device time: 48922 ns/iter; 1.0060x vs baseline; 1.0007x over previous
import jax
import jax.numpy as jnp
from jax import lax
from jax.experimental import pallas as pl
from jax.experimental.pallas import tpu as pltpu

N_DEV = 8
M = 1024
N = 1024
CHUNK = M // N_DEV
NSTREAM = 8
BAND = CHUNK // NSTREAM
SIGMA = (1, -1, 1, -1, 1, -1, 1, -1)
BANDOF = (0, 4, 1, 5, 2, 6, 3, 7)
SLOT_DEPTH = 3
SEM_DEPTH = 2
T_RS = N_DEV - 1
T_TOT = 2 * (N_DEV - 1)


def kernel(x, w_mat):
    def body(
        x_ref,
        w_ref,
        out_ref,
        pacc_ref,
        sendbuf_ref,
        slots_ref,
        send_sems,
        recv_sems,
    ):
        my = lax.axis_index("i")

        def mod8(v):
            return lax.rem(v + 16, N_DEV)

        def perm(j):
            return jnp.where(j < 4, j, 11 - j)

        my_r = perm(my)
        tgt = [perm(mod8(my_r + SIGMA[k])) for k in range(NSTREAM)]

        def band_row(ring_j, k):
            return perm(ring_j) * CHUNK + BANDOF[k] * BAND

        def gemm_rows(r, nrows):
            pacc_ref[pl.ds(r, nrows), :] = jnp.dot(
                x_ref[pl.ds(r, nrows), :],
                w_ref[...],
                preferred_element_type=jnp.float32,
            )

        def gemm_chunk(ring_j):
            gemm_rows(perm(ring_j) * CHUNK, CHUNK)

        def make(k, t):
            sig = SIGMA[k]
            if t == 0:
                src = pacc_ref.at[pl.ds(band_row(my_r, k), BAND), :]
                dst = slots_ref.at[k, 0]
            elif t < T_RS:
                src = sendbuf_ref.at[k]
                dst = slots_ref.at[k, t % SLOT_DEPTH]
            else:
                c = mod8(my_r + sig * (N_DEV - t))
                r = band_row(c, k)
                src = out_ref.at[pl.ds(r, BAND), :]
                dst = out_ref.at[pl.ds(r, BAND), :]
            return pltpu.make_async_remote_copy(
                src_ref=src,
                dst_ref=dst,
                send_sem=send_sems.at[k, t % SEM_DEPTH],
                recv_sem=recv_sems.at[k, t % SEM_DEPTH],
                device_id=(tgt[k],),
                device_id_type=pl.DeviceIdType.MESH,
            )

        barrier = pltpu.get_barrier_semaphore()
        for nbr in (tgt[0], tgt[1]):
            pl.semaphore_signal(
                barrier,
                inc=1,
                device_id=(nbr,),
                device_id_type=pl.DeviceIdType.MESH,
            )
        half = NSTREAM // 2 * BAND
        gemm_rows(my * CHUNK, half)
        pl.semaphore_wait(barrier, 2)
        for k in range(0, NSTREAM, 2):
            make(k, 0).start()
        gemm_rows(my * CHUNK + half, half)
        for k in range(1, NSTREAM, 2):
            make(k, 0).start()
        gemm_chunk(mod8(my_r + 1))
        gemm_chunk(mod8(my_r - 1))

        for t in range(1, T_TOT):
            for k in range(NSTREAM):
                sig = SIGMA[k]
                make(k, t - 1).wait()
                if t < T_RS:
                    c = mod8(my_r - sig * t)
                    sendbuf_ref[k] = (
                        pacc_ref[pl.ds(band_row(c, k), BAND), :]
                        + slots_ref[k, (t - 1) % SLOT_DEPTH]
                    )
                elif t == T_RS:
                    c = mod8(my_r + sig)
                    r = band_row(c, k)
                    acc = (
                        pacc_ref[pl.ds(r, BAND), :]
                        + slots_ref[k, (t - 1) % SLOT_DEPTH]
                    )
                    y = acc * jax.nn.sigmoid(acc)
                    out_ref[pl.ds(r, BAND), :] = y
                make(k, t).start()
            if t < 4:
                offs = [t + 1, -(t + 1)] if t < 3 else [4]
                for off in offs:
                    gemm_chunk(mod8(my_r + off))

        for k in range(NSTREAM):
            make(k, T_TOT - 1).wait()

    return pl.pallas_call(
        body,
        out_shape=jax.ShapeDtypeStruct((M, N), jnp.float32),
        in_specs=[
            pl.BlockSpec(memory_space=pltpu.VMEM),
            pl.BlockSpec(memory_space=pltpu.VMEM),
        ],
        out_specs=pl.BlockSpec(memory_space=pltpu.VMEM),
        scratch_shapes=[
            pltpu.VMEM((M, N), jnp.float32),
            pltpu.VMEM((NSTREAM, BAND, N), jnp.float32),
            pltpu.VMEM((NSTREAM, SLOT_DEPTH, BAND, N), jnp.float32),
            pltpu.SemaphoreType.DMA((NSTREAM, SEM_DEPTH)),
            pltpu.SemaphoreType.DMA((NSTREAM, SEM_DEPTH)),
        ],
        compiler_params=pltpu.CompilerParams(collective_id=0),
    )(x, w_mat)
